# Initial kernel scaffold; baseline (speedup 1.0000x reference)
import jax
import jax.numpy as jnp
from jax.experimental import pallas as pl


def kernel(x, edge_index, W1, b1, W2, b2):
    raise NotImplementedError("write your pallas kernel here")


# speedup vs baseline: 9.1980x; 9.1980x over previous
import functools

import jax
import jax.numpy as jnp
from jax import lax
from jax.experimental import pallas as pl
from jax.experimental.pallas import tpu as pltpu
from jax.experimental.pallas import tpu_sc as plsc

N = 10000
E = 320000
D1 = 128
D2 = 16

NC = 2
NS = 16
NW = NC * NS
EPW = E // NW
B = 80
STEPS = EPW // B
RPT = 624
RPT_LAST = N - (NS - 1) * RPT


def _make_sc_agg(D):
    mesh = plsc.VectorSubcoreMesh(
        core_axis_name="c", subcore_axis_name="s", num_cores=NC, num_subcores=NS
    )

    @functools.partial(
        pl.kernel,
        out_type=jax.ShapeDtypeStruct((NC, N, D), jnp.float32),
        mesh=mesh,
        scratch_types=[
            pltpu.VMEM((STEPS, B), jnp.int32),
            pltpu.VMEM((STEPS, B), jnp.int32),
            pltpu.VMEM((B, D), jnp.float32),
            pltpu.VMEM_SHARED((N, D), jnp.float32),
            pltpu.SemaphoreType.DMA,
        ],
        compiler_params=pltpu.CompilerParams(use_tc_tiling_on_sc=False),
    )
    def sc_agg(y_hbm, src_hbm, dst_hbm, zero_hbm, out_hbm,
               src_v, dst_v, rows_v, acc, sem):
        c = lax.axis_index("c")
        s = lax.axis_index("s")
        wid = s * NC + c
        row0 = s * RPT

        @pl.when(s < NS - 1)
        def _():
            pltpu.sync_copy(zero_hbm.at[pl.ds(row0, RPT)],
                            acc.at[pl.ds(row0, RPT)])

        @pl.when(s == NS - 1)
        def _():
            pltpu.sync_copy(zero_hbm.at[pl.ds(row0, RPT_LAST)],
                            acc.at[pl.ds(row0, RPT_LAST)])

        pltpu.sync_copy(src_hbm.at[wid], src_v)
        pltpu.sync_copy(dst_hbm.at[wid], dst_v)
        plsc.subcore_barrier()

        def body(j, carry):
            pltpu.async_copy(y_hbm.at[src_v.at[j]], rows_v, sem).wait()
            pltpu.sync_copy(rows_v, acc.at[dst_v.at[j]], add=True)
            return carry

        lax.fori_loop(0, STEPS, body, 0)
        plsc.subcore_barrier()

        @pl.when(s < NS - 1)
        def _():
            pltpu.sync_copy(acc.at[pl.ds(row0, RPT)],
                            out_hbm.at[c, pl.ds(row0, RPT)])

        @pl.when(s == NS - 1)
        def _():
            pltpu.sync_copy(acc.at[pl.ds(row0, RPT_LAST)],
                            out_hbm.at[c, pl.ds(row0, RPT_LAST)])

    return sc_agg


_sc_agg_d1 = _make_sc_agg(D1)
_sc_agg_d2 = _make_sc_agg(D2)


def _mid_body(p_ref, b1_ref, w1_ref, w2_ref, o_ref):
    agg = p_ref[0] + p_ref[1]
    h = jnp.maximum(
        jnp.dot(agg, w1_ref[...], preferred_element_type=jnp.float32)
        + b1_ref[...], 0.0)
    o_ref[...] = jnp.dot(h, w2_ref[...], preferred_element_type=jnp.float32)


def _fin_body(q_ref, b2_ref, o_ref):
    o_ref[...] = q_ref[0] + q_ref[1] + b2_ref[...]


def kernel(x, edge_index, W1, b1, W2, b2):
    ei = edge_index.astype(jnp.int32)
    src3 = ei[0].reshape(NW, STEPS, B)
    dst3 = ei[1].reshape(NW, STEPS, B)
    z1 = jnp.zeros((N, D1), jnp.float32)
    z2 = jnp.zeros((N, D2), jnp.float32)

    p = _sc_agg_d1(x, src3, dst3, z1)
    y2 = pl.pallas_call(
        _mid_body,
        out_shape=jax.ShapeDtypeStruct((N, D2), jnp.float32),
    )(p, b1.reshape(1, D1), W1, W2)
    q = _sc_agg_d2(y2, src3, dst3, z2)
    out = pl.pallas_call(
        _fin_body,
        out_shape=jax.ShapeDtypeStruct((N, D2), jnp.float32),
    )(q, b2.reshape(1, D2))
    return out


# speedup vs baseline: 16.5391x; 1.7981x over previous
import functools

import jax
import jax.numpy as jnp
from jax import lax
from jax.experimental import pallas as pl
from jax.experimental.pallas import tpu as pltpu
from jax.experimental.pallas import tpu_sc as plsc

N = 10000
E = 320000
D1 = 128
D2 = 16

NC = 2
NS = 16
NW = NC * NS
EPW = E // NW
CFG = {128: (40, 250, 5), 16: (80, 125, 5)}
RPT = 624
RPT_LAST = N - (NS - 1) * RPT


def _make_sc_agg(D):
    B, STEPS, R = CFG[D]
    GROUPS = STEPS // R
    mesh = plsc.VectorSubcoreMesh(
        core_axis_name="c", subcore_axis_name="s", num_cores=NC, num_subcores=NS
    )

    @functools.partial(
        pl.kernel,
        out_type=jax.ShapeDtypeStruct((NC, N, D), jnp.float32),
        mesh=mesh,
        scratch_types=[
            pltpu.VMEM((STEPS, B), jnp.int32),
            pltpu.VMEM((STEPS, B), jnp.int32),
            [pltpu.VMEM((B, D), jnp.float32) for _ in range(R)],
            pltpu.VMEM_SHARED((N, D), jnp.float32),
            pltpu.SemaphoreType.DMA((R,)),
            pltpu.SemaphoreType.DMA((R,)),
        ],
        compiler_params=pltpu.CompilerParams(use_tc_tiling_on_sc=False),
    )
    def sc_agg(y_hbm, src_hbm, dst_hbm, zero_hbm, out_hbm,
               src_v, dst_v, rows, acc, gsem, ssem):
        c = lax.axis_index("c")
        s = lax.axis_index("s")
        wid = s * NC + c
        row0 = s * RPT

        @pl.when(s < NS - 1)
        def _():
            pltpu.sync_copy(zero_hbm.at[pl.ds(row0, RPT)],
                            acc.at[pl.ds(row0, RPT)])

        @pl.when(s == NS - 1)
        def _():
            pltpu.sync_copy(zero_hbm.at[pl.ds(row0, RPT_LAST)],
                            acc.at[pl.ds(row0, RPT_LAST)])

        pltpu.sync_copy(src_hbm.at[wid], src_v)
        pltpu.sync_copy(dst_hbm.at[wid], dst_v)
        plsc.subcore_barrier()

        for b in range(R):
            pltpu.async_copy(y_hbm.at[src_v.at[b]], rows[b], gsem.at[b])

        def group(g, carry):
            descs = []
            for b in range(R):
                j = g * R + b
                pltpu.make_async_copy(
                    y_hbm.at[src_v.at[j]], rows[b], gsem.at[b]).wait()
                descs.append(pltpu.async_copy(
                    rows[b], acc.at[dst_v.at[j]], ssem.at[b], add=True))
            for b in range(R):
                descs[b].wait()

                @pl.when(g + 1 < GROUPS)
                def _():
                    jn = (g + 1) * R + b
                    pltpu.async_copy(
                        y_hbm.at[src_v.at[jn]], rows[b], gsem.at[b])
            return carry

        lax.fori_loop(0, GROUPS, group, 0)
        plsc.subcore_barrier()

        @pl.when(s < NS - 1)
        def _():
            pltpu.sync_copy(acc.at[pl.ds(row0, RPT)],
                            out_hbm.at[c, pl.ds(row0, RPT)])

        @pl.when(s == NS - 1)
        def _():
            pltpu.sync_copy(acc.at[pl.ds(row0, RPT_LAST)],
                            out_hbm.at[c, pl.ds(row0, RPT_LAST)])

    return sc_agg


_sc_agg_d1 = _make_sc_agg(D1)
_sc_agg_d2 = _make_sc_agg(D2)


def _mid_body(p_ref, b1_ref, w1_ref, w2_ref, o_ref):
    agg = p_ref[0] + p_ref[1]
    h = jnp.maximum(
        jnp.dot(agg, w1_ref[...], preferred_element_type=jnp.float32)
        + b1_ref[...], 0.0)
    o_ref[...] = jnp.dot(h, w2_ref[...], preferred_element_type=jnp.float32)


def _fin_body(q_ref, b2_ref, o_ref):
    o_ref[...] = q_ref[0] + q_ref[1] + b2_ref[...]


def kernel(x, edge_index, W1, b1, W2, b2):
    ei = edge_index.astype(jnp.int32)
    b_1, s_1, _ = CFG[D1]
    b_2, s_2, _ = CFG[D2]
    src1 = ei[0].reshape(NW, s_1, b_1)
    dst1 = ei[1].reshape(NW, s_1, b_1)
    src2 = ei[0].reshape(NW, s_2, b_2)
    dst2 = ei[1].reshape(NW, s_2, b_2)
    z1 = jnp.zeros((N, D1), jnp.float32)
    z2 = jnp.zeros((N, D2), jnp.float32)

    p = _sc_agg_d1(x, src1, dst1, z1)
    y2 = pl.pallas_call(
        _mid_body,
        out_shape=jax.ShapeDtypeStruct((N, D2), jnp.float32),
    )(p, b1.reshape(1, D1), W1, W2)
    q = _sc_agg_d2(y2, src2, dst2, z2)
    out = pl.pallas_call(
        _fin_body,
        out_shape=jax.ShapeDtypeStruct((N, D2), jnp.float32),
    )(q, b2.reshape(1, D2))
    return out
